# baseline (speedup 1.0000x reference)
import functools

import jax
import jax.numpy as jnp
from jax import lax
from jax.experimental import pallas as pl
from jax.experimental.pallas import tpu as pltpu
from jax.experimental.pallas import tpu_sc as plsc

G = 8

L = 16
_IN_BOUNDS = jax.lax.GatherScatterMode.PROMISE_IN_BOUNDS


_GDN = lax.GatherDimensionNumbers(
    offset_dims=(), collapsed_slice_dims=(0,), start_index_map=(0,))


def _shuffle(x, idx):
    return lax.gather(x, idx[:, None], _GDN, (1,), mode=_IN_BOUNDS)


def _butterfly_idx():
    lanes = lax.iota(jnp.int32, L)
    return [lanes ^ (1 << p) for p in range(4)]


def _hsum(x, bidx):
    for idx in bidx:
        x = x + _shuffle(x, idx)
    return x


def _hmax(x, bidx):
    for idx in bidx:
        x = jnp.maximum(x, _shuffle(x, idx))
    return x


def _sc_node_compute(nb_ref, nf_all, wcT, asrc_v, adst_v, li, d, k, c):
    nch = d // L
    lanes = lax.iota(jnp.int32, L)
    bidx = _butterfly_idx()

    acc = nf_all[li, pl.ds(0, L)] * asrc_v[pl.ds(0, L)]
    for j in range(1, nch):
        acc = acc + nf_all[li, pl.ds(j * L, L)] * asrc_v[pl.ds(j * L, L)]
    e_src = _hsum(acc, bidx)

    ev = [jnp.zeros((L,), jnp.float32) for _ in range(k // L)]
    for kk in range(k):
        a = nb_ref[kk, pl.ds(0, L)] * adst_v[pl.ds(0, L)]
        for j in range(1, nch):
            a = a + nb_ref[kk, pl.ds(j * L, L)] * adst_v[pl.ds(j * L, L)]
        e_k = _hsum(a, bidx) + e_src
        e_k = jnp.where(e_k >= 0.0, e_k, 0.2 * e_k)
        g = kk // L
        ev[g] = jnp.where(lanes == (kk % L), e_k, ev[g])

    m = _hmax(ev[0], bidx)
    for g in range(1, k // L):
        m = jnp.maximum(m, _hmax(ev[g], bidx))
    ex = [jnp.exp(v - m) for v in ev]
    s = _hsum(ex[0], bidx)
    for g in range(1, k // L):
        s = s + _hsum(ex[g], bidx)
    alpha = [v / s for v in ex]

    xagg = [jnp.zeros((L,), jnp.float32) for _ in range(nch)]
    for kk in range(k):
        idx = jnp.full((L,), kk % L, jnp.int32)
        a_k = _shuffle(alpha[kk // L], idx)
        for j in range(nch):
            xagg[j] = xagg[j] + a_k * nb_ref[kk, pl.ds(j * L, L)]

    outv = []
    for cg in range(c // L):
        o = jnp.zeros((L,), jnp.float32)
        for cl in range(L):
            cc = cg * L + cl
            a = xagg[0] * wcT[cc, pl.ds(0, L)]
            for j in range(1, nch):
                a = a + xagg[j] * wcT[cc, pl.ds(j * L, L)]
            o = jnp.where(lanes == cl, _hsum(a, bidx), o)
        outv.append(o)
    return outv


def _sc_gat_kernel(npw, n, off, nf_hbm, nb_hbm, asrc_hbm, adst_hbm, wcT_hbm,
                   out_hbm, nf_all, asrc_v, adst_v, wcT,
                   nb0, nb1, out_all,
                   sem_in0, sem_in1):
    k, d = nb0.shape
    c = out_all.shape[1]
    info = plsc.get_sparse_core_info()
    nc = info.num_cores
    wid = lax.axis_index("s") * nc + lax.axis_index("c")
    base = off + wid * npw

    pltpu.sync_copy(nf_hbm.at[pl.ds(base, npw)], nf_all)
    pltpu.sync_copy(asrc_hbm, asrc_v)
    pltpu.sync_copy(adst_hbm, adst_v)
    pltpu.sync_copy(wcT_hbm, wcT)

    def ok(i):
        return jnp.logical_and(i < npw, base + i < n)

    @pl.when(ok(0))
    def _():
        pltpu.async_copy(nb_hbm.at[base], nb0, sem_in0)

    def compute_store(nb_ref, i):
        outv = _sc_node_compute(nb_ref, nf_all, wcT, asrc_v, adst_v,
                                i, d, k, c)
        for cg in range(c // L):
            out_all[i, pl.ds(cg * L, L)] = outv[cg]

    def body(i2, carry):
        e0 = 2 * i2
        e1 = e0 + 1
        e2 = e0 + 2

        @pl.when(ok(e1))
        def _():
            pltpu.async_copy(nb_hbm.at[base + e1], nb1, sem_in1)

        @pl.when(ok(e0))
        def _():
            pltpu.make_async_copy(nb_hbm.at[0], nb0, sem_in0).wait()
            compute_store(nb0, e0)

        @pl.when(ok(e2))
        def _():
            pltpu.async_copy(nb_hbm.at[base + e2], nb0, sem_in0)

        @pl.when(ok(e1))
        def _():
            pltpu.make_async_copy(nb_hbm.at[0], nb1, sem_in1).wait()
            compute_store(nb1, e1)
        return carry

    lax.fori_loop(0, (npw + 1) // 2, body, 0, unroll=False)

    pltpu.sync_copy(out_all, out_hbm.at[pl.ds(base - off, npw)])


def _sc_gat(node_feature, neighbor, asrc_v, adst_v, wcT, off, n_sc):
    n, d = node_feature.shape
    k = neighbor.shape[1]
    c = wcT.shape[0]
    nw = 32
    npw = n_sc // nw

    mesh = plsc.VectorSubcoreMesh(core_axis_name="c", subcore_axis_name="s")
    kfn = functools.partial(
        pl.kernel,
        out_type=jax.ShapeDtypeStruct((n_sc, c), jnp.float32),
        mesh=mesh,
        scratch_types=[
            pltpu.VMEM((npw, d), jnp.float32),
            pltpu.VMEM((d,), jnp.float32),
            pltpu.VMEM((d,), jnp.float32),
            pltpu.VMEM((c, d), jnp.float32),
            pltpu.VMEM((k, d), jnp.float32),
            pltpu.VMEM((k, d), jnp.float32),
            pltpu.VMEM((npw, c), jnp.float32),
            pltpu.SemaphoreType.DMA,
            pltpu.SemaphoreType.DMA,
        ],
    )(functools.partial(_sc_gat_kernel, npw, n, off))
    return kfn(node_feature, neighbor, asrc_v, adst_v, wcT)


def _gat_block_kernel(nf_ref, nb_ref, asrc_ref, adst_ref, wc_ref, out_ref):
    nf = nf_ref[...]
    nb = nb_ref[...]
    asrc_v = asrc_ref[...]
    adst_v = adst_ref[...]
    wc = wc_ref[...]
    b, k, d = nb.shape

    e_src = jnp.sum(nf * asrc_v, axis=1, keepdims=True)
    nb3 = nb.reshape(b // G, G * k, d)
    adst_bc = jnp.broadcast_to(adst_v, (b // G, G, d))
    e3 = jax.lax.dot_general(
        adst_bc, nb3, (((2,), (2,)), ((0,), (0,))),
        preferred_element_type=jnp.float32)
    e2 = e3.reshape(b, G * k)
    lane0 = jax.lax.broadcasted_iota(jnp.int32, (b, G * k), 1) // k
    row0 = jax.lax.broadcasted_iota(jnp.int32, (b, G * k), 0) % G
    e_sel = jnp.where(lane0 == row0, e2, 0.0)
    e_dst = jnp.sum(e_sel.reshape(b, G, k), axis=1)
    e = e_src + e_dst
    e = jnp.where(e >= 0, e, 0.2 * e)
    e_max = jnp.max(e, axis=1, keepdims=True)
    ex = jnp.exp(e - e_max)
    alpha = ex / jnp.sum(ex, axis=1, keepdims=True)

    tiled = jnp.tile(alpha, (1, G))
    lane = jax.lax.broadcasted_iota(jnp.int32, (b, G * k), 1) // k
    row = jax.lax.broadcasted_iota(jnp.int32, (b, G * k), 0) % G
    adiag = jnp.where(lane == row, tiled, 0.0)

    a3 = adiag.reshape(b // G, G, G * k)
    xagg = jax.lax.dot_general(
        a3, nb3, (((2,), (1,)), ((0,), (0,))),
        preferred_element_type=jnp.float32).reshape(b, d)

    out_ref[...] = jnp.dot(xagg, wc, preferred_element_type=jnp.float32)


def _tc_gat(node_feature, neighbor, asrc_v, adst_v, wc, block, n_tc):
    n, d = node_feature.shape
    k = neighbor.shape[1]
    c = wc.shape[1]
    grid = (pl.cdiv(n_tc, block),)
    return pl.pallas_call(
        _gat_block_kernel,
        grid=grid,
        in_specs=[
            pl.BlockSpec((block, d), lambda i: (i, 0)),
            pl.BlockSpec((block, k, d), lambda i: (i, 0, 0)),
            pl.BlockSpec((1, d), lambda i: (0, 0)),
            pl.BlockSpec((1, d), lambda i: (0, 0)),
            pl.BlockSpec((d, c), lambda i: (0, 0)),
        ],
        out_specs=pl.BlockSpec((block, c), lambda i: (i, 0)),
        out_shape=jax.ShapeDtypeStruct((n_tc, c), jnp.float32),
        compiler_params=pltpu.CompilerParams(
            dimension_semantics=("parallel",),
        ),
    )(node_feature, neighbor, asrc_v, adst_v, wc)


N_SC = 512
TC_BLOCK = 1000


def kernel(node_feature, neighbor_nodes_feature, W, a_src, a_dst, weight):
    n, d = node_feature.shape
    asrc_v = (W @ a_src[0]).reshape(1, d)
    adst_v = (W @ a_dst[0]).reshape(1, d)
    wc = W @ weight

    n_tc = n - N_SC
    sc_out = _sc_gat(node_feature, neighbor_nodes_feature,
                     asrc_v.reshape(d), adst_v.reshape(d), wc.T,
                     n_tc, N_SC)
    tc_out = _tc_gat(node_feature, neighbor_nodes_feature,
                     asrc_v, adst_v, wc, TC_BLOCK, n_tc)
    return jnp.concatenate([tc_out, sc_out], axis=0)


# speedup vs baseline: 1.0133x; 1.0133x over previous
import functools

import jax
import jax.numpy as jnp
from jax import lax
from jax.experimental import pallas as pl
from jax.experimental.pallas import tpu as pltpu
from jax.experimental.pallas import tpu_sc as plsc

G = 8

L = 16
_IN_BOUNDS = jax.lax.GatherScatterMode.PROMISE_IN_BOUNDS


_GDN = lax.GatherDimensionNumbers(
    offset_dims=(), collapsed_slice_dims=(0,), start_index_map=(0,))


def _shuffle(x, idx):
    return lax.gather(x, idx[:, None], _GDN, (1,), mode=_IN_BOUNDS)


def _butterfly_idx():
    lanes = lax.iota(jnp.int32, L)
    return [lanes ^ (1 << p) for p in range(4)]


def _hsum(x, bidx):
    for idx in bidx:
        x = x + _shuffle(x, idx)
    return x


def _hmax(x, bidx):
    for idx in bidx:
        x = jnp.maximum(x, _shuffle(x, idx))
    return x


def _sc_node_compute(nb_ref, nf_all, prm, li, d, k, c):
    nch = d // L
    lanes = lax.iota(jnp.int32, L)
    bidx = _butterfly_idx()

    acc = nf_all[li, pl.ds(0, L)] * prm[c, pl.ds(0, L)]
    for j in range(1, nch):
        acc = acc + nf_all[li, pl.ds(j * L, L)] * prm[c, pl.ds(j * L, L)]
    e_src = _hsum(acc, bidx)

    ev = [jnp.zeros((L,), jnp.float32) for _ in range(k // L)]
    for kk in range(k):
        a = nb_ref[kk, pl.ds(0, L)] * prm[c + 1, pl.ds(0, L)]
        for j in range(1, nch):
            a = a + nb_ref[kk, pl.ds(j * L, L)] * prm[c + 1, pl.ds(j * L, L)]
        e_k = _hsum(a, bidx) + e_src
        e_k = jnp.where(e_k >= 0.0, e_k, 0.2 * e_k)
        g = kk // L
        ev[g] = jnp.where(lanes == (kk % L), e_k, ev[g])

    m = _hmax(ev[0], bidx)
    for g in range(1, k // L):
        m = jnp.maximum(m, _hmax(ev[g], bidx))
    ex = [jnp.exp(v - m) for v in ev]
    s = _hsum(ex[0], bidx)
    for g in range(1, k // L):
        s = s + _hsum(ex[g], bidx)
    alpha = [v / s for v in ex]

    xagg = [jnp.zeros((L,), jnp.float32) for _ in range(nch)]
    for kk in range(k):
        idx = jnp.full((L,), kk % L, jnp.int32)
        a_k = _shuffle(alpha[kk // L], idx)
        for j in range(nch):
            xagg[j] = xagg[j] + a_k * nb_ref[kk, pl.ds(j * L, L)]

    outv = []
    for cg in range(c // L):
        o = jnp.zeros((L,), jnp.float32)
        for cl in range(L):
            cc = cg * L + cl
            a = xagg[0] * prm[cc, pl.ds(0, L)]
            for j in range(1, nch):
                a = a + xagg[j] * prm[cc, pl.ds(j * L, L)]
            o = jnp.where(lanes == cl, _hsum(a, bidx), o)
        outv.append(o)
    return outv


def _sc_gat_kernel(npw, n, off, nf_hbm, nb_hbm, prm_hbm,
                   out_hbm, nf_all, prm,
                   nb0, nb1, out_all,
                   sem_in0, sem_in1):
    k, d = nb0.shape
    c = out_all.shape[1]
    info = plsc.get_sparse_core_info()
    nc = info.num_cores
    wid = lax.axis_index("s") * nc + lax.axis_index("c")
    base = off + wid * npw

    pltpu.sync_copy(nf_hbm.at[pl.ds(base, npw)], nf_all)
    pltpu.sync_copy(prm_hbm, prm)

    def ok(i):
        return jnp.logical_and(i < npw, base + i < n)

    @pl.when(ok(0))
    def _():
        pltpu.async_copy(nb_hbm.at[base], nb0, sem_in0)

    def compute_store(nb_ref, i):
        outv = _sc_node_compute(nb_ref, nf_all, prm, i, d, k, c)
        for cg in range(c // L):
            out_all[i, pl.ds(cg * L, L)] = outv[cg]

    def body(i2, carry):
        e0 = 2 * i2
        e1 = e0 + 1
        e2 = e0 + 2

        @pl.when(ok(e1))
        def _():
            pltpu.async_copy(nb_hbm.at[base + e1], nb1, sem_in1)

        @pl.when(ok(e0))
        def _():
            pltpu.make_async_copy(nb_hbm.at[0], nb0, sem_in0).wait()
            compute_store(nb0, e0)

        @pl.when(ok(e2))
        def _():
            pltpu.async_copy(nb_hbm.at[base + e2], nb0, sem_in0)

        @pl.when(ok(e1))
        def _():
            pltpu.make_async_copy(nb_hbm.at[0], nb1, sem_in1).wait()
            compute_store(nb1, e1)
        return carry

    lax.fori_loop(0, (npw + 1) // 2, body, 0, unroll=False)

    pltpu.sync_copy(out_all, out_hbm.at[pl.ds(base - off, npw)])


def _sc_gat(node_feature, neighbor, asrc_v, adst_v, wcT, off, n_sc):
    n, d = node_feature.shape
    k = neighbor.shape[1]
    c = wcT.shape[0]
    nw = 32
    npw = n_sc // nw
    prm = jnp.concatenate([wcT, asrc_v[None, :], adst_v[None, :]], axis=0)

    mesh = plsc.VectorSubcoreMesh(core_axis_name="c", subcore_axis_name="s")
    kfn = functools.partial(
        pl.kernel,
        out_type=jax.ShapeDtypeStruct((n_sc, c), jnp.float32),
        mesh=mesh,
        scratch_types=[
            pltpu.VMEM((npw, d), jnp.float32),
            pltpu.VMEM((c + 2, d), jnp.float32),
            pltpu.VMEM((k, d), jnp.float32),
            pltpu.VMEM((k, d), jnp.float32),
            pltpu.VMEM((npw, c), jnp.float32),
            pltpu.SemaphoreType.DMA,
            pltpu.SemaphoreType.DMA,
        ],
    )(functools.partial(_sc_gat_kernel, npw, n, off))
    return kfn(node_feature, neighbor, prm)


def _gat_block_kernel(nf_ref, nb_ref, asrc_ref, adst_ref, wc_ref, out_ref):
    nf = nf_ref[...]
    nb = nb_ref[...]
    asrc_v = asrc_ref[...]
    adst_v = adst_ref[...]
    wc = wc_ref[...]
    b, k, d = nb.shape

    e_src = jnp.sum(nf * asrc_v, axis=1, keepdims=True)
    nb3 = nb.reshape(b // G, G * k, d)
    adst_bc = jnp.broadcast_to(adst_v, (b // G, G, d))
    e3 = jax.lax.dot_general(
        adst_bc, nb3, (((2,), (2,)), ((0,), (0,))),
        preferred_element_type=jnp.float32)
    e2 = e3.reshape(b, G * k)
    lane0 = jax.lax.broadcasted_iota(jnp.int32, (b, G * k), 1) // k
    row0 = jax.lax.broadcasted_iota(jnp.int32, (b, G * k), 0) % G
    e_sel = jnp.where(lane0 == row0, e2, 0.0)
    e_dst = jnp.sum(e_sel.reshape(b, G, k), axis=1)
    e = e_src + e_dst
    e = jnp.where(e >= 0, e, 0.2 * e)
    e_max = jnp.max(e, axis=1, keepdims=True)
    ex = jnp.exp(e - e_max)
    alpha = ex / jnp.sum(ex, axis=1, keepdims=True)

    tiled = jnp.tile(alpha, (1, G))
    lane = jax.lax.broadcasted_iota(jnp.int32, (b, G * k), 1) // k
    row = jax.lax.broadcasted_iota(jnp.int32, (b, G * k), 0) % G
    adiag = jnp.where(lane == row, tiled, 0.0)

    a3 = adiag.reshape(b // G, G, G * k)
    xagg = jax.lax.dot_general(
        a3, nb3, (((2,), (1,)), ((0,), (0,))),
        preferred_element_type=jnp.float32).reshape(b, d)

    out_ref[...] = jnp.dot(xagg, wc, preferred_element_type=jnp.float32)


def _tc_gat(node_feature, neighbor, asrc_v, adst_v, wc, block, n_tc):
    n, d = node_feature.shape
    k = neighbor.shape[1]
    c = wc.shape[1]
    grid = (pl.cdiv(n_tc, block),)
    return pl.pallas_call(
        _gat_block_kernel,
        grid=grid,
        in_specs=[
            pl.BlockSpec((block, d), lambda i: (i, 0)),
            pl.BlockSpec((block, k, d), lambda i: (i, 0, 0)),
            pl.BlockSpec((1, d), lambda i: (0, 0)),
            pl.BlockSpec((1, d), lambda i: (0, 0)),
            pl.BlockSpec((d, c), lambda i: (0, 0)),
        ],
        out_specs=pl.BlockSpec((block, c), lambda i: (i, 0)),
        out_shape=jax.ShapeDtypeStruct((n_tc, c), jnp.float32),
        compiler_params=pltpu.CompilerParams(
            dimension_semantics=("parallel",),
        ),
    )(node_feature, neighbor, asrc_v, adst_v, wc)


N_SC = 512
TC_BLOCK = 1000


def kernel(node_feature, neighbor_nodes_feature, W, a_src, a_dst, weight):
    n, d = node_feature.shape
    asrc_v = (W @ a_src[0]).reshape(1, d)
    adst_v = (W @ a_dst[0]).reshape(1, d)
    wc = W @ weight

    n_tc = n - N_SC
    sc_out = _sc_gat(node_feature, neighbor_nodes_feature,
                     asrc_v.reshape(d), adst_v.reshape(d), wc.T,
                     n_tc, N_SC)
    tc_out = _tc_gat(node_feature, neighbor_nodes_feature,
                     asrc_v, adst_v, wc, TC_BLOCK, n_tc)
    return jnp.concatenate([tc_out, sc_out], axis=0)


# speedup vs baseline: 1.0469x; 1.0332x over previous
import functools

import jax
import jax.numpy as jnp
from jax import lax
from jax.experimental import pallas as pl
from jax.experimental.pallas import tpu as pltpu
from jax.experimental.pallas import tpu_sc as plsc

G = 8

L = 16
_IN_BOUNDS = jax.lax.GatherScatterMode.PROMISE_IN_BOUNDS


_GDN = lax.GatherDimensionNumbers(
    offset_dims=(), collapsed_slice_dims=(0,), start_index_map=(0,))


def _shuffle(x, idx):
    return lax.gather(x, idx[:, None], _GDN, (1,), mode=_IN_BOUNDS)


def _butterfly_idx():
    lanes = lax.iota(jnp.int32, L)
    return [lanes ^ (1 << p) for p in range(4)]


def _hsum(x, bidx):
    for idx in bidx:
        x = x + _shuffle(x, idx)
    return x


def _hmax(x, bidx):
    for idx in bidx:
        x = jnp.maximum(x, _shuffle(x, idx))
    return x


def _sc_node_compute(nb_ref, nf_all, prm, li, d, k, c):
    nch = d // L
    lanes = lax.iota(jnp.int32, L)
    bidx = _butterfly_idx()

    acc = nf_all[li, pl.ds(0, L)] * prm[c, pl.ds(0, L)]
    for j in range(1, nch):
        acc = acc + nf_all[li, pl.ds(j * L, L)] * prm[c, pl.ds(j * L, L)]
    e_src = _hsum(acc, bidx)

    ev = [jnp.zeros((L,), jnp.float32) for _ in range(k // L)]
    for kk in range(k):
        a = nb_ref[kk, pl.ds(0, L)] * prm[c + 1, pl.ds(0, L)]
        for j in range(1, nch):
            a = a + nb_ref[kk, pl.ds(j * L, L)] * prm[c + 1, pl.ds(j * L, L)]
        e_k = _hsum(a, bidx) + e_src
        e_k = jnp.where(e_k >= 0.0, e_k, 0.2 * e_k)
        g = kk // L
        ev[g] = jnp.where(lanes == (kk % L), e_k, ev[g])

    m = _hmax(ev[0], bidx)
    for g in range(1, k // L):
        m = jnp.maximum(m, _hmax(ev[g], bidx))
    ex = [jnp.exp(v - m) for v in ev]
    s = _hsum(ex[0], bidx)
    for g in range(1, k // L):
        s = s + _hsum(ex[g], bidx)
    alpha = [v / s for v in ex]

    xagg = [jnp.zeros((L,), jnp.float32) for _ in range(nch)]
    for kk in range(k):
        idx = jnp.full((L,), kk % L, jnp.int32)
        a_k = _shuffle(alpha[kk // L], idx)
        for j in range(nch):
            xagg[j] = xagg[j] + a_k * nb_ref[kk, pl.ds(j * L, L)]

    outv = []
    for cg in range(c // L):
        o = jnp.zeros((L,), jnp.float32)
        for cl in range(L):
            cc = cg * L + cl
            a = xagg[0] * prm[cc, pl.ds(0, L)]
            for j in range(1, nch):
                a = a + xagg[j] * prm[cc, pl.ds(j * L, L)]
            o = jnp.where(lanes == cl, _hsum(a, bidx), o)
        outv.append(o)
    return outv


def _sc_gat_kernel(npw, n, off, nf_hbm, nb_hbm, prm_hbm,
                   out_hbm, nf_all, prm,
                   nb0, nb1, out_all,
                   sem_in0, sem_in1):
    k, d = nb0.shape
    c = out_all.shape[1]
    info = plsc.get_sparse_core_info()
    nc = info.num_cores
    wid = lax.axis_index("s") * nc + lax.axis_index("c")
    base = off + wid * npw

    pltpu.sync_copy(nf_hbm.at[pl.ds(base, npw)], nf_all)
    pltpu.sync_copy(prm_hbm, prm)

    def ok(i):
        return jnp.logical_and(i < npw, base + i < n)

    @pl.when(ok(0))
    def _():
        pltpu.async_copy(nb_hbm.at[base], nb0, sem_in0)

    def compute_store(nb_ref, i):
        outv = _sc_node_compute(nb_ref, nf_all, prm, i, d, k, c)
        for cg in range(c // L):
            out_all[i, pl.ds(cg * L, L)] = outv[cg]

    def body(i2, carry):
        e0 = 2 * i2
        e1 = e0 + 1
        e2 = e0 + 2

        @pl.when(ok(e1))
        def _():
            pltpu.async_copy(nb_hbm.at[base + e1], nb1, sem_in1)

        @pl.when(ok(e0))
        def _():
            pltpu.make_async_copy(nb_hbm.at[0], nb0, sem_in0).wait()
            compute_store(nb0, e0)

        @pl.when(ok(e2))
        def _():
            pltpu.async_copy(nb_hbm.at[base + e2], nb0, sem_in0)

        @pl.when(ok(e1))
        def _():
            pltpu.make_async_copy(nb_hbm.at[0], nb1, sem_in1).wait()
            compute_store(nb1, e1)
        return carry

    lax.fori_loop(0, (npw + 1) // 2, body, 0, unroll=False)

    pltpu.sync_copy(out_all, out_hbm.at[pl.ds(base - off, npw)])


def _sc_gat(node_feature, neighbor, asrc_v, adst_v, wcT, off, n_sc):
    n, d = node_feature.shape
    k = neighbor.shape[1]
    c = wcT.shape[0]
    nw = 32
    npw = n_sc // nw
    prm = jnp.concatenate([wcT, asrc_v[None, :], adst_v[None, :]], axis=0)

    mesh = plsc.VectorSubcoreMesh(core_axis_name="c", subcore_axis_name="s")
    kfn = functools.partial(
        pl.kernel,
        out_type=jax.ShapeDtypeStruct((n_sc, c), jnp.float32),
        mesh=mesh,
        scratch_types=[
            pltpu.VMEM((npw, d), jnp.float32),
            pltpu.VMEM((c + 2, d), jnp.float32),
            pltpu.VMEM((k, d), jnp.float32),
            pltpu.VMEM((k, d), jnp.float32),
            pltpu.VMEM((npw, c), jnp.float32),
            pltpu.SemaphoreType.DMA,
            pltpu.SemaphoreType.DMA,
        ],
    )(functools.partial(_sc_gat_kernel, npw, n, off))
    return kfn(node_feature, neighbor, prm)


def _gat_block_kernel(nf_ref, nb_ref, asrc_ref, adst_ref, wc_ref, out_ref):
    nf = nf_ref[...]
    nb = nb_ref[...]
    asrc_v = asrc_ref[...]
    adst_v = adst_ref[...]
    wc = wc_ref[...]
    b, k, d = nb.shape

    e_src = jnp.sum(nf * asrc_v, axis=1, keepdims=True)
    nb3 = nb.reshape(b // G, G * k, d)
    adst_bc = jnp.broadcast_to(adst_v, (b // G, G, d))
    e3 = jax.lax.dot_general(
        adst_bc, nb3, (((2,), (2,)), ((0,), (0,))),
        preferred_element_type=jnp.float32)
    e2 = e3.reshape(b, G * k)
    lane0 = jax.lax.broadcasted_iota(jnp.int32, (b, G * k), 1) // k
    row0 = jax.lax.broadcasted_iota(jnp.int32, (b, G * k), 0) % G
    e_sel = jnp.where(lane0 == row0, e2, 0.0)
    e_dst = jnp.sum(e_sel.reshape(b, G, k), axis=1)
    e = e_src + e_dst
    e = jnp.where(e >= 0, e, 0.2 * e)
    e_max = jnp.max(e, axis=1, keepdims=True)
    ex = jnp.exp(e - e_max)
    alpha = ex / jnp.sum(ex, axis=1, keepdims=True)

    tiled = jnp.tile(alpha, (1, G))
    lane = jax.lax.broadcasted_iota(jnp.int32, (b, G * k), 1) // k
    row = jax.lax.broadcasted_iota(jnp.int32, (b, G * k), 0) % G
    adiag = jnp.where(lane == row, tiled, 0.0)

    a3 = adiag.reshape(b // G, G, G * k)
    xagg = jax.lax.dot_general(
        a3, nb3, (((2,), (1,)), ((0,), (0,))),
        preferred_element_type=jnp.float32).reshape(b, d)

    out_ref[...] = jnp.dot(xagg, wc, preferred_element_type=jnp.float32)


def _tc_gat(node_feature, neighbor, asrc_v, adst_v, wc, block, n_tc):
    n, d = node_feature.shape
    k = neighbor.shape[1]
    c = wc.shape[1]
    grid = (pl.cdiv(n_tc, block),)
    return pl.pallas_call(
        _gat_block_kernel,
        grid=grid,
        in_specs=[
            pl.BlockSpec((block, d), lambda i: (i, 0)),
            pl.BlockSpec((block, k, d), lambda i: (i, 0, 0)),
            pl.BlockSpec((1, d), lambda i: (0, 0)),
            pl.BlockSpec((1, d), lambda i: (0, 0)),
            pl.BlockSpec((d, c), lambda i: (0, 0)),
        ],
        out_specs=pl.BlockSpec((block, c), lambda i: (i, 0)),
        out_shape=jax.ShapeDtypeStruct((n_tc, c), jnp.float32),
        compiler_params=pltpu.CompilerParams(
            dimension_semantics=("parallel",),
        ),
    )(node_feature, neighbor, asrc_v, adst_v, wc)


N_SC = 512
TC_BLOCK = 1192


def kernel(node_feature, neighbor_nodes_feature, W, a_src, a_dst, weight):
    n, d = node_feature.shape
    asrc_v = (W @ a_src[0]).reshape(1, d)
    adst_v = (W @ a_dst[0]).reshape(1, d)
    wc = W @ weight

    n_tc = n - N_SC
    sc_out = _sc_gat(node_feature, neighbor_nodes_feature,
                     asrc_v.reshape(d), adst_v.reshape(d), wc.T,
                     n_tc, N_SC)
    tc_out = _tc_gat(node_feature, neighbor_nodes_feature,
                     asrc_v, adst_v, wc, TC_BLOCK, n_tc)
    return jnp.concatenate([tc_out, sc_out], axis=0)
